# Initial kernel scaffold; baseline (speedup 1.0000x reference)
import jax
import jax.numpy as jnp
from jax.experimental import pallas as pl


def kernel(input, w1s, w2s):
    raise NotImplementedError("write your pallas kernel here")


# speedup vs baseline: 3.8309x; 3.8309x over previous
import functools
import math

import jax
import jax.numpy as jnp
from jax import lax
from jax.experimental import pallas as pl
from jax.experimental.pallas import tpu as pltpu


def _fff_block_kernel(x_ref, w1_ref, w2_ref, out_ref, *, depth, n_pad):
    x = x_ref[...]
    m = x.shape[0]
    logits = lax.dot_general(
        x, w1_ref[...], (((1,), (1,)), ((), ())),
        preferred_element_type=jnp.float32,
        precision=lax.Precision.HIGHEST,
    )

    p = jnp.zeros((m, 1), jnp.int32)
    pieces = []
    for lvl in range(depth):
        w = 1 << lvl
        base = w - 1
        ls = lax.slice(logits, (0, base), (m, base + w))
        col = lax.broadcasted_iota(jnp.int32, (m, w), 1)
        sel = col == p
        logit = jnp.sum(jnp.where(sel, ls, 0.0), axis=1, keepdims=True)
        act = jax.nn.gelu(logit)
        pieces.append(jnp.where(sel, act, 0.0))
        p = 2 * p + (logit > 0.0).astype(jnp.int32)
    n_nodes = (1 << depth) - 1
    if n_pad > n_nodes:
        pieces.append(jnp.zeros((m, n_pad - n_nodes), jnp.float32))
    acts = jnp.concatenate(pieces, axis=1).astype(jnp.bfloat16)

    out_ref[...] = lax.dot_general(
        acts, w2_ref[...], (((1,), (0,)), ((), ())),
        preferred_element_type=jnp.float32,
    )


@jax.jit
def kernel(input, w1s, w2s):
    tokens, nin = input.shape
    n_nodes, nout = w2s.shape
    depth = int(math.log2(n_nodes + 1))
    n_pad = n_nodes + 1

    w1p = jnp.concatenate([w1s, jnp.zeros((n_pad - n_nodes, nin), w1s.dtype)])
    w2p = jnp.concatenate([w2s, jnp.zeros((n_pad - n_nodes, nout), w2s.dtype)])
    w2p = w2p.astype(jnp.bfloat16)

    m = 256
    grid = (tokens // m,)
    return pl.pallas_call(
        functools.partial(_fff_block_kernel, depth=depth, n_pad=n_pad),
        grid=grid,
        in_specs=[
            pl.BlockSpec((m, nin), lambda i: (i, 0)),
            pl.BlockSpec((n_pad, nin), lambda i: (0, 0)),
            pl.BlockSpec((n_pad, nout), lambda i: (0, 0)),
        ],
        out_specs=pl.BlockSpec((m, nout), lambda i: (i, 0)),
        out_shape=jax.ShapeDtypeStruct((tokens, nout), jnp.float32),
    )(input, w1p, w2p)


# speedup vs baseline: 5.6839x; 1.4837x over previous
import functools
import math

import jax
import jax.numpy as jnp
from jax import lax
from jax.experimental import pallas as pl
from jax.experimental.pallas import tpu as pltpu


def _fff_block_kernel(x_ref, w1h_ref, w1l_ref, w2_ref, out_ref, *, depth, n_pad):
    x = x_ref[...]
    m = x.shape[0]
    xh = x.astype(jnp.bfloat16)
    xl = (x - xh.astype(jnp.float32)).astype(jnp.bfloat16)
    dn = (((1,), (1,)), ((), ()))
    w1h = w1h_ref[...]
    logits = lax.dot_general(xh, w1h, dn, preferred_element_type=jnp.float32)
    logits += lax.dot_general(xl, w1h, dn, preferred_element_type=jnp.float32)
    logits += lax.dot_general(xh, w1l_ref[...], dn,
                              preferred_element_type=jnp.float32)

    p = jnp.zeros((m, 1), jnp.int32)
    pieces = []
    for lvl in range(depth):
        w = 1 << lvl
        base = w - 1
        ls = lax.slice(logits, (0, base), (m, base + w))
        col = lax.broadcasted_iota(jnp.int32, (m, w), 1)
        sel = col == p
        logit = jnp.sum(jnp.where(sel, ls, 0.0), axis=1, keepdims=True)
        act = jax.nn.gelu(logit)
        pieces.append(jnp.where(sel, act, 0.0))
        p = 2 * p + (logit > 0.0).astype(jnp.int32)
    n_nodes = (1 << depth) - 1
    if n_pad > n_nodes:
        pieces.append(jnp.zeros((m, n_pad - n_nodes), jnp.float32))
    acts = jnp.concatenate(pieces, axis=1).astype(jnp.bfloat16)

    out_ref[...] = lax.dot_general(
        acts, w2_ref[...], (((1,), (0,)), ((), ())),
        preferred_element_type=jnp.float32,
    )


@jax.jit
def kernel(input, w1s, w2s):
    tokens, nin = input.shape
    n_nodes, nout = w2s.shape
    depth = int(math.log2(n_nodes + 1))
    n_pad = n_nodes + 1

    w1p = jnp.concatenate([w1s, jnp.zeros((n_pad - n_nodes, nin), w1s.dtype)])
    w1h = w1p.astype(jnp.bfloat16)
    w1l = (w1p - w1h.astype(jnp.float32)).astype(jnp.bfloat16)
    w2p = jnp.concatenate([w2s, jnp.zeros((n_pad - n_nodes, nout), w2s.dtype)])
    w2p = w2p.astype(jnp.bfloat16)

    m = 256
    grid = (tokens // m,)
    return pl.pallas_call(
        functools.partial(_fff_block_kernel, depth=depth, n_pad=n_pad),
        grid=grid,
        in_specs=[
            pl.BlockSpec((m, nin), lambda i: (i, 0)),
            pl.BlockSpec((n_pad, nin), lambda i: (0, 0)),
            pl.BlockSpec((n_pad, nin), lambda i: (0, 0)),
            pl.BlockSpec((n_pad, nout), lambda i: (0, 0)),
        ],
        out_specs=pl.BlockSpec((m, nout), lambda i: (i, 0)),
        out_shape=jax.ShapeDtypeStruct((tokens, nout), jnp.float32),
    )(input, w1h, w1l, w2p)
